# Initial kernel scaffold; baseline (speedup 1.0000x reference)
import jax
import jax.numpy as jnp
from jax.experimental import pallas as pl


def kernel(x, W_embed, b_embed, bn_gamma, bn_beta, bn_mean, bn_var, W_last, b_last, adj_vals, adj2_vals, adj_rows, adj_cols, adj2_rows, adj2_cols):
    raise NotImplementedError("write your pallas kernel here")


# speedup vs baseline: 6.1710x; 6.1710x over previous
import functools

import jax
import jax.numpy as jnp
from jax import lax
from jax.experimental import pallas as pl
from jax.experimental.pallas import tpu as pltpu
from jax.experimental.pallas import tpu_sc as plsc

N = 10000
NH = N // 2
NHP = NH + 8
BLK = 128
WB = 8
L = 16


def _align8_up(v):
    return ((v + 7) >> 3) << 3


def _align8_dn(v):
    return (v >> 3) << 3


def _make_spmm(EA, EB, D):
    D2 = 2 * D
    mesh = plsc.VectorSubcoreMesh(core_axis_name="c", subcore_axis_name="s")

    @functools.partial(
        pl.kernel,
        out_type=jax.ShapeDtypeStruct((2, N, D), jnp.float32),
        mesh=mesh,
        compiler_params=pltpu.CompilerParams(use_tc_tiling_on_sc=False),
        scratch_types=[
            pltpu.VMEM_SHARED((2 * NHP, D), jnp.float32),
            pltpu.VMEM((BLK,), jnp.int32),
            pltpu.VMEM((BLK,), jnp.int32),
            pltpu.VMEM((BLK,), jnp.float32),
            pltpu.VMEM((BLK,), jnp.int32),
            pltpu.VMEM((BLK, D), jnp.float32),
            pltpu.VMEM((WB, D), jnp.float32),
            pltpu.VMEM((WB, D), jnp.float32),
            pltpu.VMEM((WB, D), jnp.float32),
            pltpu.VMEM((D2,), jnp.float32),
            pltpu.VMEM((D2,), jnp.float32),
            pltpu.VMEM((L,), jnp.int32),
            pltpu.VMEM((L,), jnp.int32),
            pltpu.SemaphoreType.DMA,
        ],
    )
    def spmm(feat, rowsa, colsa, valsa, rowsb, colsb, valsb, scale, shift,
             bnda, bndb, out,
             acc, colv, rowv, valv, rloc, gbuf, zbuf, obufa, obufb, sv, tv,
             bva, bvb, gsem):
        c = lax.axis_index("c")
        sid = lax.axis_index("s")
        row_lo = c * NH

        pltpu.sync_copy(scale, sv)
        pltpu.sync_copy(shift, tv)
        pltpu.sync_copy(bnda, bva)
        pltpu.sync_copy(bndb, bvb)

        z16 = jnp.zeros((L,), jnp.float32)
        for i in range(WB):
            for j in range(D // L):
                zbuf[i, pl.ds(j * L, L)] = z16

        @pl.loop(sid, (2 * NHP) // WB, step=16)
        def _zero(blk):
            pltpu.sync_copy(zbuf, acc.at[pl.ds(blk * WB, WB)])

        plsc.subcore_barrier()

        iota = lax.iota(jnp.int32, L)

        def do_edges(rows_hbm, cols_hbm, vals_hbm, bv_ref, e_main, acc_base):
            bnd = bv_ref[...][0]
            lo = jnp.where(c == 0, 0, _align8_dn(bnd))
            hi = jnp.where(c == 0, _align8_up(bnd), e_main)
            chunk = _align8_up((hi - lo + 15) >> 4)
            t_lo = lo + sid * chunk
            t_hi = jnp.minimum(t_lo + chunk, hi)

            @pl.loop(t_lo, t_hi, step=BLK)
            def _block(e0):
                e0 = pl.multiple_of(e0, 8)
                limit = t_hi - e0
                pltpu.sync_copy(cols_hbm.at[pl.ds(e0, BLK)], colv)
                pltpu.sync_copy(rows_hbm.at[pl.ds(e0, BLK)], rowv)
                pltpu.sync_copy(vals_hbm.at[pl.ds(e0, BLK)], valv)
                pltpu.async_copy(feat.at[colv], gbuf, gsem).wait()

                @pl.loop(0, BLK // L)
                def _group(g):
                    g16 = g * L
                    r16 = rowv[pl.ds(g16, L)]
                    v16 = valv[pl.ds(g16, L)]
                    rl = r16 - row_lo
                    m = (rl >= 0) & (rl < NH) & ((g16 + iota) < limit)
                    v16 = jnp.where(m, v16, jnp.zeros((L,), jnp.float32))
                    rl = jnp.clip(rl, 0, NH - 1) + acc_base
                    rloc[pl.ds(g16, L)] = rl
                    for i in range(L):
                        v = v16[i]
                        for j in range(D // L):
                            sl = pl.ds(j * L, L)
                            gbuf[g16 + i, sl] = gbuf[g16 + i, sl] * v

                pltpu.sync_copy(gbuf, acc.at[rloc], add=True)

        do_edges(rowsa, colsa, valsa, bva, EA, 0)
        do_edges(rowsb, colsb, valsb, bvb, EB, NHP)
        plsc.subcore_barrier()

        @pl.loop(sid, NH // WB, step=16)
        def _wb(blk):
            r0 = blk * WB
            pltpu.sync_copy(acc.at[pl.ds(r0, WB)], obufa)
            pltpu.sync_copy(acc.at[pl.ds(NHP + r0, WB)], obufb)
            for i in range(WB):
                for j in range(D // L):
                    sl = pl.ds(j * L, L)
                    slb = pl.ds(D + j * L, L)
                    obufa[i, sl] = obufa[i, sl] * sv[sl] + tv[sl]
                    obufb[i, sl] = obufb[i, sl] * sv[slb] + tv[slb]
            gr = row_lo + r0
            pltpu.sync_copy(obufa, out.at[0, pl.ds(gr, WB)])
            pltpu.sync_copy(obufb, out.at[1, pl.ds(gr, WB)])

    return spmm


def _prep(rows, cols, vals):
    e = rows.shape[0]
    em = ((e + 7) // 8) * 8
    ln = em + 2048
    rows = rows.astype(jnp.int32)
    cols = cols.astype(jnp.int32)
    vals = vals.astype(jnp.float32)
    rows_p = jnp.full((ln,), N - 1, jnp.int32).at[:e].set(rows)
    cols_p = jnp.zeros((ln,), jnp.int32).at[:e].set(cols)
    vals_p = jnp.zeros((ln,), jnp.float32).at[:e].set(vals)
    bnd = jnp.searchsorted(rows, NH).astype(jnp.int32)
    bv = jnp.zeros((16,), jnp.int32).at[0].set(bnd)
    return rows_p, cols_p, vals_p, bv, em


def _embed(x, w, b):
    bn = 2000

    def body(x_ref, w_ref, b_ref, o_ref):
        o_ref[...] = jnp.maximum(
            jnp.dot(x_ref[...], w_ref[...],
                    preferred_element_type=jnp.float32) + b_ref[...], 0.0)

    return pl.pallas_call(
        body,
        grid=(N // bn,),
        in_specs=[
            pl.BlockSpec((bn, 128), lambda i: (i, 0)),
            pl.BlockSpec((128, 64), lambda i: (0, 0)),
            pl.BlockSpec((1, 64), lambda i: (0, 0)),
        ],
        out_specs=pl.BlockSpec((bn, 64), lambda i: (i, 0)),
        out_shape=jax.ShapeDtypeStruct((N, 64), jnp.float32),
    )(x, w, b.reshape(1, 64))


def _final(h, c1, c2, w_last, b_last):
    bn = 2000
    w0 = w_last[0:64]
    w1 = w_last[64:192]
    w2 = w_last[192:448]

    def body(h_ref, c1_ref, c2_ref, w0_ref, w1_ref, w2_ref, b_ref, o_ref):
        acc = jnp.dot(h_ref[...], w0_ref[...],
                      preferred_element_type=jnp.float32)
        acc += jnp.dot(c1_ref[...], w1_ref[...],
                       preferred_element_type=jnp.float32)
        acc += jnp.dot(c2_ref[...], w2_ref[...],
                       preferred_element_type=jnp.float32)
        o_ref[...] = acc + b_ref[...]

    return pl.pallas_call(
        body,
        grid=(N // bn,),
        in_specs=[
            pl.BlockSpec((bn, 64), lambda i: (i, 0)),
            pl.BlockSpec((bn, 128), lambda i: (i, 0)),
            pl.BlockSpec((bn, 256), lambda i: (i, 0)),
            pl.BlockSpec((64, 128), lambda i: (0, 0)),
            pl.BlockSpec((128, 128), lambda i: (0, 0)),
            pl.BlockSpec((256, 128), lambda i: (0, 0)),
            pl.BlockSpec((1, 128), lambda i: (0, 0)),
        ],
        out_specs=pl.BlockSpec((bn, 128), lambda i: (i, 0)),
        out_shape=jax.ShapeDtypeStruct((N, 128), jnp.float32),
    )(h, c1, c2, w0, w1, w2, b_last.reshape(1, 128))


def kernel(x, W_embed, b_embed, bn_gamma, bn_beta, bn_mean, bn_var, W_last,
           b_last, adj_vals, adj2_vals, adj_rows, adj_cols, adj2_rows,
           adj2_cols):
    h = _embed(x, W_embed, b_embed)

    s = bn_gamma * lax.rsqrt(bn_var + 1e-5)
    t = bn_beta - bn_mean * s

    rowsa, colsa, valsa, bnda, ea = _prep(adj_rows, adj_cols, adj_vals)
    rowsb, colsb, valsb, bndb, eb = _prep(adj2_rows, adj2_cols, adj2_vals)

    o1 = _make_spmm(ea, eb, 64)(h, rowsa, colsa, valsa, rowsb, colsb, valsb,
                                s, t, bnda, bndb)
    c1 = jnp.concatenate([o1[0], o1[1]], axis=1)

    ident_s = jnp.ones((256,), jnp.float32)
    ident_t = jnp.zeros((256,), jnp.float32)
    o2 = _make_spmm(ea, eb, 128)(c1, rowsa, colsa, valsa, rowsb, colsb,
                                 valsb, ident_s, ident_t, bnda, bndb)
    c2 = jnp.concatenate([o2[0], o2[1]], axis=1)

    return _final(h, c1, c2, W_last, b_last)


# speedup vs baseline: 18.5512x; 3.0062x over previous
import functools

import jax
import jax.numpy as jnp
from jax import lax
from jax.experimental import pallas as pl
from jax.experimental.pallas import tpu as pltpu
from jax.experimental.pallas import tpu_sc as plsc

N = 10000
NH = N // 2
NHP = NH + 8
BLK = 128
WB = 8
L = 16


def _align8_up(v):
    return ((v + 7) >> 3) << 3


def _align8_dn(v):
    return (v >> 3) << 3


def _make_spmm(EA, EB, D):
    D2 = 2 * D
    mesh = plsc.VectorSubcoreMesh(core_axis_name="c", subcore_axis_name="s")

    @functools.partial(
        pl.kernel,
        out_type=jax.ShapeDtypeStruct((2, N, D), jnp.float32),
        mesh=mesh,
        compiler_params=pltpu.CompilerParams(use_tc_tiling_on_sc=False),
        scratch_types=[
            pltpu.VMEM_SHARED((2 * NHP, D), jnp.float32),
            pltpu.VMEM((2, BLK), jnp.int32),
            pltpu.VMEM((2, BLK), jnp.int32),
            pltpu.VMEM((2, BLK), jnp.float32),
            pltpu.VMEM((2, BLK), jnp.int32),
            pltpu.VMEM((2, BLK), jnp.float32),
            pltpu.VMEM((2, BLK, D), jnp.float32),
            pltpu.VMEM((WB, D), jnp.float32),
            pltpu.VMEM((WB, D), jnp.float32),
            pltpu.VMEM((WB, D), jnp.float32),
            pltpu.VMEM((D2,), jnp.float32),
            pltpu.VMEM((D2,), jnp.float32),
            pltpu.VMEM((L,), jnp.int32),
            pltpu.VMEM((L,), jnp.int32),
            pltpu.SemaphoreType.DMA,
            pltpu.SemaphoreType.DMA,
            pltpu.SemaphoreType.DMA,
            pltpu.SemaphoreType.DMA,
            pltpu.SemaphoreType.DMA,
            pltpu.SemaphoreType.DMA,
        ],
    )
    def spmm(feat, rowsa, colsa, valsa, rowsb, colsb, valsb, scale, shift,
             bnda, bndb, out,
             acc, colv, rowv, valv, rloc, vmsk, gbuf, zbuf, obufa, obufb,
             sv, tv, bva, bvb, esem0, esem1, gsem0, gsem1, ssem0, ssem1):
        c = lax.axis_index("c")
        sid = lax.axis_index("s")
        row_lo = c * NH

        pltpu.sync_copy(scale, sv)
        pltpu.sync_copy(shift, tv)
        pltpu.sync_copy(bnda, bva)
        pltpu.sync_copy(bndb, bvb)

        z16 = jnp.zeros((L,), jnp.float32)
        for i in range(WB):
            for j in range(D // L):
                zbuf[i, pl.ds(j * L, L)] = z16

        @pl.loop(sid, (2 * NHP) // WB, step=16)
        def _zero(blk):
            pltpu.sync_copy(zbuf, acc.at[pl.ds(blk * WB, WB)])

        plsc.subcore_barrier()

        iota = lax.iota(jnp.int32, L)
        esem = (esem0, esem1)
        gsem = (gsem0, gsem1)
        ssem = (ssem0, ssem1)
        fzero = jnp.zeros((L,), jnp.float32)

        def do_edges(rows_hbm, cols_hbm, vals_hbm, bv_ref, e_main, acc_base):
            bnd = bv_ref[...][0]
            lo = jnp.where(c == 0, 0, _align8_dn(bnd))
            hi = jnp.where(c == 0, _align8_up(bnd), e_main)
            chunk = _align8_up((hi - lo + 15) >> 4)
            t_lo = lo + sid * chunk
            t_hi = jnp.minimum(t_lo + chunk, hi)
            nbp = jnp.maximum((chunk + 2 * BLK - 1) >> 8, 1)

            def edge_descs(s, e0):
                e0 = pl.multiple_of(e0, 8)
                return (
                    pltpu.make_async_copy(cols_hbm.at[pl.ds(e0, BLK)],
                                          colv.at[s], esem[s]),
                    pltpu.make_async_copy(rows_hbm.at[pl.ds(e0, BLK)],
                                          rowv.at[s], esem[s]),
                    pltpu.make_async_copy(vals_hbm.at[pl.ds(e0, BLK)],
                                          valv.at[s], esem[s]),
                )

            def gather_desc(s):
                return pltpu.make_async_copy(feat.at[colv.at[s]], gbuf.at[s],
                                             gsem[s])

            def scat_desc(s):
                return pltpu.make_async_copy(gbuf.at[s], acc.at[rloc.at[s]],
                                             ssem[s])

            def mask_block(s, e0):
                limit = t_hi - e0

                @pl.loop(0, BLK // L, unroll=8)
                def _group(g):
                    g16 = g * L
                    r16 = rowv[s, pl.ds(g16, L)]
                    v16 = valv[s, pl.ds(g16, L)]
                    rl = r16 - row_lo
                    m = (rl >= 0) & (rl < NH) & ((g16 + iota) < limit)
                    rloc[s, pl.ds(g16, L)] = jnp.clip(rl, 0, NH - 1) + acc_base
                    vmsk[s, pl.ds(g16, L)] = jnp.where(m, v16, fzero)

            def scale_block(s):
                @pl.loop(0, BLK // L)
                def _group(g):
                    g16 = g * L
                    v16 = vmsk[s, pl.ds(g16, L)]
                    for i in range(L):
                        v = v16[i]
                        for j in range(D // L):
                            sl = pl.ds(j * L, L)
                            gbuf[s, g16 + i, sl] = gbuf[s, g16 + i, sl] * v

            for d in edge_descs(0, t_lo):
                d.start()
            for d in edge_descs(1, t_lo + BLK):
                d.start()

            @pl.loop(0, nbp)
            def _pair(i):
                base = t_lo + i * (2 * BLK)
                for b in (0, 1):
                    o = 1 - b
                    e_k = base + b * BLK
                    for d in edge_descs(b, e_k):
                        d.wait()

                    @pl.when(i > 0)
                    def _():
                        scat_desc(b).wait()

                    gather_desc(b).start()
                    mask_block(b, e_k)

                    def tail():
                        gather_desc(o).wait()
                        if b == 0:
                            for d in edge_descs(o, e_k + BLK):
                                d.start()
                        else:
                            @pl.when(i < nbp - 1)
                            def _():
                                for d in edge_descs(o, e_k + BLK):
                                    d.start()
                        scale_block(o)
                        scat_desc(o).start(add=True)

                    if b == 1:
                        tail()
                    else:
                        @pl.when(i > 0)
                        def _():
                            tail()

            scat_desc(0).wait()
            gather_desc(1).wait()
            scale_block(1)
            scat_desc(1).start(add=True)
            scat_desc(1).wait()

        do_edges(rowsa, colsa, valsa, bva, EA, 0)
        do_edges(rowsb, colsb, valsb, bvb, EB, NHP)
        plsc.subcore_barrier()

        @pl.loop(sid, NH // WB, step=16)
        def _wb(blk):
            r0 = blk * WB
            pltpu.sync_copy(acc.at[pl.ds(r0, WB)], obufa)
            pltpu.sync_copy(acc.at[pl.ds(NHP + r0, WB)], obufb)
            for i in range(WB):
                for j in range(D // L):
                    sl = pl.ds(j * L, L)
                    slb = pl.ds(D + j * L, L)
                    obufa[i, sl] = obufa[i, sl] * sv[sl] + tv[sl]
                    obufb[i, sl] = obufb[i, sl] * sv[slb] + tv[slb]
            gr = row_lo + r0
            pltpu.sync_copy(obufa, out.at[0, pl.ds(gr, WB)])
            pltpu.sync_copy(obufb, out.at[1, pl.ds(gr, WB)])

    return spmm


def _prep(rows, cols, vals):
    e = rows.shape[0]
    em = ((e + 7) // 8) * 8
    ln = em + 2048
    rows = rows.astype(jnp.int32)
    cols = cols.astype(jnp.int32)
    vals = vals.astype(jnp.float32)
    rows_p = jnp.full((ln,), N - 1, jnp.int32).at[:e].set(rows)
    cols_p = jnp.zeros((ln,), jnp.int32).at[:e].set(cols)
    vals_p = jnp.zeros((ln,), jnp.float32).at[:e].set(vals)
    bnd = jnp.searchsorted(rows, NH).astype(jnp.int32)
    bv = jnp.zeros((16,), jnp.int32).at[0].set(bnd)
    return rows_p, cols_p, vals_p, bv, em


def _embed(x, w, b):
    bn = 2000

    def body(x_ref, w_ref, b_ref, o_ref):
        o_ref[...] = jnp.maximum(
            jnp.dot(x_ref[...], w_ref[...],
                    preferred_element_type=jnp.float32) + b_ref[...], 0.0)

    return pl.pallas_call(
        body,
        grid=(N // bn,),
        in_specs=[
            pl.BlockSpec((bn, 128), lambda i: (i, 0)),
            pl.BlockSpec((128, 64), lambda i: (0, 0)),
            pl.BlockSpec((1, 64), lambda i: (0, 0)),
        ],
        out_specs=pl.BlockSpec((bn, 64), lambda i: (i, 0)),
        out_shape=jax.ShapeDtypeStruct((N, 64), jnp.float32),
    )(x, w, b.reshape(1, 64))


def _final(h, c1, c2, w_last, b_last):
    bn = 2000
    w0 = w_last[0:64]
    w1 = w_last[64:192]
    w2 = w_last[192:448]

    def body(h_ref, c1_ref, c2_ref, w0_ref, w1_ref, w2_ref, b_ref, o_ref):
        acc = jnp.dot(h_ref[...], w0_ref[...],
                      preferred_element_type=jnp.float32)
        acc += jnp.dot(c1_ref[...], w1_ref[...],
                       preferred_element_type=jnp.float32)
        acc += jnp.dot(c2_ref[...], w2_ref[...],
                       preferred_element_type=jnp.float32)
        o_ref[...] = acc + b_ref[...]

    return pl.pallas_call(
        body,
        grid=(N // bn,),
        in_specs=[
            pl.BlockSpec((bn, 64), lambda i: (i, 0)),
            pl.BlockSpec((bn, 128), lambda i: (i, 0)),
            pl.BlockSpec((bn, 256), lambda i: (i, 0)),
            pl.BlockSpec((64, 128), lambda i: (0, 0)),
            pl.BlockSpec((128, 128), lambda i: (0, 0)),
            pl.BlockSpec((256, 128), lambda i: (0, 0)),
            pl.BlockSpec((1, 128), lambda i: (0, 0)),
        ],
        out_specs=pl.BlockSpec((bn, 128), lambda i: (i, 0)),
        out_shape=jax.ShapeDtypeStruct((N, 128), jnp.float32),
    )(h, c1, c2, w0, w1, w2, b_last.reshape(1, 128))


def kernel(x, W_embed, b_embed, bn_gamma, bn_beta, bn_mean, bn_var, W_last,
           b_last, adj_vals, adj2_vals, adj_rows, adj_cols, adj2_rows,
           adj2_cols):
    h = _embed(x, W_embed, b_embed)

    s = bn_gamma * lax.rsqrt(bn_var + 1e-5)
    t = bn_beta - bn_mean * s

    rowsa, colsa, valsa, bnda, ea = _prep(adj_rows, adj_cols, adj_vals)
    rowsb, colsb, valsb, bndb, eb = _prep(adj2_rows, adj2_cols, adj2_vals)

    o1 = _make_spmm(ea, eb, 64)(h, rowsa, colsa, valsa, rowsb, colsb, valsb,
                                s, t, bnda, bndb)
    c1 = jnp.concatenate([o1[0], o1[1]], axis=1)

    ident_s = jnp.ones((256,), jnp.float32)
    ident_t = jnp.zeros((256,), jnp.float32)
    o2 = _make_spmm(ea, eb, 128)(c1, rowsa, colsa, valsa, rowsb, colsb,
                                 valsb, ident_s, ident_t, bnda, bndb)
    c2 = jnp.concatenate([o2[0], o2[1]], axis=1)

    return _final(h, c1, c2, W_last, b_last)
